# baseline (speedup 1.0000x reference)
import functools
import math

import numpy as np

import jax
import jax.numpy as jnp
from jax import lax
from jax.experimental import pallas as pl
from jax.experimental.pallas import tpu as pltpu
from jax.experimental.pallas import tpu_sc as plsc

N_NODES = 10000
N_EDGES = 320000
IN_DIM = 128
HID = 128
N_TYPES = 4
N_REL = 8
N_HEADS = 8
D_K = 16
N_LAYERS = 2
N_OUT = 16

NPAD = 10240
NB = 256
EPAD = N_EDGES + 128

NW = 32
EPW = N_EDGES // NW
CH = 16
NFULL = 625
DEN_W = NPAD * N_HEADS

HALF = NPAD // 2
ACC_ROWS = HALF + 128
ZROWS = ACC_ROWS // 16
DROWS = HALF // 16
SCH = 80
EPS = N_EDGES // 16
NCHS = EPS // SCH

def _pe_table():
    t = np.arange(128, dtype=np.float64)[:, None]
    div = np.exp(np.arange(0, HID, 2, dtype=np.float64) * (-(math.log(10000.0) / HID)))
    ang = t * div[None, :]
    pe = np.stack([np.sin(ang), np.cos(ang)], axis=-1).reshape(128, HID)
    return jnp.asarray(pe / math.sqrt(HID), jnp.float32)


def _blockdiag(a):
    out = jnp.zeros(a.shape[:-3] + (HID, HID), jnp.float32)
    for h in range(N_HEADS):
        out = out.at[..., h * D_K:(h + 1) * D_K, h * D_K:(h + 1) * D_K].set(a[..., h, :, :])
    return out


def _adapt_body(nf, oh, W, b, out):
    x = nf[...]
    ohv = oh[...]
    acc = jnp.zeros((NB, HID), jnp.float32)
    for t in range(N_TYPES):
        y = jnp.tanh(jnp.dot(x, W[t], preferred_element_type=jnp.float32)
                     + b[t][None, :])
        acc = acc + ohv[:, t][:, None] * y
    out[...] = acc


def _adapt(nf_p, oh_p, adapt_W, adapt_b):
    return pl.pallas_call(
        _adapt_body,
        grid=(NPAD // NB,),
        in_specs=[
            pl.BlockSpec((NB, IN_DIM), lambda i: (i, 0)),
            pl.BlockSpec((NB, N_TYPES), lambda i: (i, 0)),
            pl.BlockSpec((N_TYPES, IN_DIM, HID), lambda i: (0, 0, 0)),
            pl.BlockSpec((N_TYPES, HID), lambda i: (0, 0)),
        ],
        out_specs=pl.BlockSpec((NB, HID), lambda i: (i, 0)),
        out_shape=jax.ShapeDtypeStruct((NPAD, HID), jnp.float32),
    )(nf_p, oh_p, adapt_W, adapt_b)


def _tables_body(pe, rw, rb, Wk, Wv, Amsg, rk_out, rv_out):
    rte = jnp.dot(pe[...], rw[...], preferred_element_type=jnp.float32) + rb[0][None, :]
    for t in range(N_TYPES):
        rk_out[t] = jnp.dot(rte, Wk[t], preferred_element_type=jnp.float32)
        rv = jnp.dot(rte, Wv[t], preferred_element_type=jnp.float32)
        for r in range(N_REL):
            rv_out[t, :, r, :] = jnp.dot(rv, Amsg[r], preferred_element_type=jnp.float32)


def _tables(pe, rw, rb2, Wk_l, Wv_l, Amsg):
    rk, rv = pl.pallas_call(
        _tables_body,
        grid=(1,),
        in_specs=[
            pl.BlockSpec((128, HID), lambda i: (0, 0)),
            pl.BlockSpec((HID, HID), lambda i: (0, 0)),
            pl.BlockSpec((1, HID), lambda i: (0, 0)),
            pl.BlockSpec((N_TYPES, HID, HID), lambda i: (0, 0, 0)),
            pl.BlockSpec((N_TYPES, HID, HID), lambda i: (0, 0, 0)),
            pl.BlockSpec((N_REL, HID, HID), lambda i: (0, 0, 0)),
        ],
        out_specs=[
            pl.BlockSpec((N_TYPES, 128, HID), lambda i: (0, 0, 0)),
            pl.BlockSpec((N_TYPES, 128, N_REL, HID), lambda i: (0, 0, 0, 0)),
        ],
        out_shape=[
            jax.ShapeDtypeStruct((N_TYPES, 128, HID), jnp.float32),
            jax.ShapeDtypeStruct((N_TYPES, 128, N_REL, HID), jnp.float32),
        ],
    )(pe, rw, rb2, Wk_l, Wv_l, Amsg)
    return rk.reshape(N_TYPES * 128, HID), rv.reshape(N_TYPES * 128 * N_REL, HID)


def _nodepre_body(x, oh, Wk, bk, Wq, bq, Wv, bv, Aatt, Amsg, kn_out, qr_out, vr_out):
    xv = x[...]
    ohv = oh[...]

    def tlin(W, b):
        acc = jnp.zeros((NB, HID), jnp.float32)
        for t in range(N_TYPES):
            acc = acc + ohv[:, t][:, None] * (
                jnp.dot(xv, W[t], preferred_element_type=jnp.float32)
                + b[t][None, :])
        return acc

    K = tlin(Wk, bk)
    Q = tlin(Wq, bq)
    V = tlin(Wv, bv)
    kn_out[...] = K
    for r in range(N_REL):
        qr_out[:, r, :] = jnp.dot(Q, Aatt[r], preferred_element_type=jnp.float32)
        vr_out[:, r, :] = jnp.dot(V, Amsg[r], preferred_element_type=jnp.float32)


def _nodepre(x, oh_p, Wk_l, bk_l, Wq_l, bq_l, Wv_l, bv_l, Aatt, Amsg):
    w3 = pl.BlockSpec((N_TYPES, HID, HID), lambda i: (0, 0, 0))
    b2 = pl.BlockSpec((N_TYPES, HID), lambda i: (0, 0))
    r3 = pl.BlockSpec((N_REL, HID, HID), lambda i: (0, 0, 0))
    kn, qr, vr = pl.pallas_call(
        _nodepre_body,
        grid=(NPAD // NB,),
        in_specs=[
            pl.BlockSpec((NB, HID), lambda i: (i, 0)),
            pl.BlockSpec((NB, N_TYPES), lambda i: (i, 0)),
            w3, b2, w3, b2, w3, b2, r3, r3,
        ],
        out_specs=[
            pl.BlockSpec((NB, HID), lambda i: (i, 0)),
            pl.BlockSpec((NB, N_REL, HID), lambda i: (i, 0, 0)),
            pl.BlockSpec((NB, N_REL, HID), lambda i: (i, 0, 0)),
        ],
        out_shape=[
            jax.ShapeDtypeStruct((NPAD, HID), jnp.float32),
            jax.ShapeDtypeStruct((NPAD, N_REL, HID), jnp.float32),
            jax.ShapeDtypeStruct((NPAD, N_REL, HID), jnp.float32),
        ],
    )(x, oh_p, Wk_l, bk_l, Wq_l, bq_l, Wv_l, bv_l, Aatt, Amsg)
    return kn, qr.reshape(NPAD * N_REL, HID), vr.reshape(NPAD * N_REL, HID)


def _update_body(num_r, denp, x, oh, Wa, ba, sig, out):
    den8 = jnp.sum(denp[...], axis=0)
    i1 = lax.broadcasted_iota(jnp.int32, (N_HEADS, HID), 0)
    j1 = lax.broadcasted_iota(jnp.int32, (N_HEADS, HID), 1)
    Sm = (i1 == (j1 // D_K)).astype(jnp.float32)
    denx = jnp.dot(den8, Sm, preferred_element_type=jnp.float32) + 1e-16
    agg = num_r[...] / denx
    g = jax.nn.gelu(agg)
    ohv = oh[...]
    trans = jnp.zeros((NB, HID), jnp.float32)
    for t in range(N_TYPES):
        trans = trans + ohv[:, t][:, None] * (
            jnp.dot(g, Wa[t], preferred_element_type=jnp.float32)
            + ba[t][None, :])
    alph = jnp.dot(ohv, sig[...], preferred_element_type=jnp.float32)
    out[...] = trans * alph + x[...] * (1.0 - alph)


def _update(num, denp, x, oh_p, Wa_l, ba_l, sig_col):
    return pl.pallas_call(
        _update_body,
        grid=(NPAD // NB,),
        in_specs=[
            pl.BlockSpec((NB, HID), lambda i: (i, 0)),
            pl.BlockSpec((NW, NB, N_HEADS), lambda i: (0, i, 0)),
            pl.BlockSpec((NB, HID), lambda i: (i, 0)),
            pl.BlockSpec((NB, N_TYPES), lambda i: (i, 0)),
            pl.BlockSpec((N_TYPES, HID, HID), lambda i: (0, 0, 0)),
            pl.BlockSpec((N_TYPES, HID), lambda i: (0, 0)),
            pl.BlockSpec((N_TYPES, 1), lambda i: (0, 0)),
        ],
        out_specs=pl.BlockSpec((NB, HID), lambda i: (i, 0)),
        out_shape=jax.ShapeDtypeStruct((NPAD, HID), jnp.float32),
    )(num, denp, x, oh_p, Wa_l, ba_l, sig_col)


def _cls_body(x, W, b, out):
    logits = jnp.dot(x[...], W[...], preferred_element_type=jnp.float32) + b[0][None, :]
    m = jnp.max(logits, axis=-1, keepdims=True)
    z = logits - m
    out[...] = z - jnp.log(jnp.sum(jnp.exp(z), axis=-1, keepdims=True))


def _cls(x, cls_W, cls_b2):
    return pl.pallas_call(
        _cls_body,
        grid=(NPAD // NB,),
        in_specs=[
            pl.BlockSpec((NB, HID), lambda i: (i, 0)),
            pl.BlockSpec((HID, N_OUT), lambda i: (0, 0)),
            pl.BlockSpec((1, N_OUT), lambda i: (0, 0)),
        ],
        out_specs=pl.BlockSpec((NB, N_OUT), lambda i: (i, 0)),
        out_shape=jax.ShapeDtypeStruct((NPAD, N_OUT), jnp.float32),
    )(x, cls_W, cls_b2)


def _sc_edge_body(src_h, dst_h, ik_h, iq_h, iv_h, ir_h,
                  kn_h, rtek_h, qr_h, vr_h, rtev_h,
                  m_h, denp_h,
                  srcb0, dstb0, ikb0, iqb0, ivb0, irb0,
                  srcb1, dstb1, ikb1, iqb1, ivb1, irb1,
                  kb0, rkb0, qb0, vb0, rvb0,
                  kb1, rkb1, qb1, vb1, rvb1,
                  denf, attb, evb,
                  semI0, semI1, semG0, semG1, semS0, semS1):
    cid = lax.axis_index("c")
    sid = lax.axis_index("s")
    wid = sid * 2 + cid
    base0 = wid * EPW
    lane = lax.broadcasted_iota(jnp.int32, (16,), 0)

    idxs = ((srcb0, dstb0, ikb0, iqb0, ivb0, irb0),
            (srcb1, dstb1, ikb1, iqb1, ivb1, irb1))
    data = ((kb0, rkb0, qb0, vb0, rvb0),
            (kb1, rkb1, qb1, vb1, rvb1))
    semI = (semI0, semI1)
    semG = (semG0, semG1)
    semS = (semS0, semS1)
    ih = (src_h, dst_h, ik_h, iq_h, iv_h, ir_h)
    th = (kn_h, rtek_h, qr_h, vr_h, rtev_h)

    def fire_idx(c, p):
        for a in range(6):
            pltpu.async_copy(ih[a].at[pl.ds(base0 + c * CH, CH)], idxs[p][a], semI[p])

    def wait_idx(c, p):
        for a in range(6):
            pltpu.make_async_copy(ih[a].at[pl.ds(base0 + c * CH, CH)],
                                  idxs[p][a], semI[p]).wait()

    def gidx(p):
        return (idxs[p][0], idxs[p][2], idxs[p][3], idxs[p][4], idxs[p][5])

    def fire_gath(p):
        g = gidx(p)
        for a in range(5):
            pltpu.async_copy(th[a].at[g[a]], data[p][a], semG[p])

    def wait_gath(p):
        g = gidx(p)
        for a in range(5):
            pltpu.make_async_copy(th[a].at[g[a]], data[p][a], semG[p]).wait()

    def fire_store(c, p):
        pltpu.async_copy(data[p][0], m_h.at[pl.ds(base0 + c * CH, CH)], semS[p])

    def wait_store(c, p):
        pltpu.make_async_copy(data[p][0], m_h.at[pl.ds(base0 + c * CH, CH)], semS[p]).wait()

    hmask = lane < N_HEADS
    m15 = lane == 15
    z16 = lane * 0

    def compute(p):
        kb, rkb, qb, vb, rvb = data[p]
        dvec = idxs[p][1][pl.ds(0, 16)]
        for i in range(CH):
            for h in range(N_HEADS):
                sl = pl.ds(h * D_K, 16)
                kq = (kb[i, sl] + rkb[i, sl]) * qb[i, sl]
                cs = plsc.cumsum(kq)
                plsc.store_scatter(attb, [z16 + h], cs, mask=m15)
            attv = attb[pl.ds(0, 16)]
            ev = jnp.exp(jnp.minimum(attv, 80.0))
            evb[pl.ds(0, 16)] = ev
            plsc.addupdate_scatter(denf, [dvec[i] * N_HEADS + lane], ev, mask=hmask)
            for h in range(N_HEADS):
                sl = pl.ds(h * D_K, 16)
                eh = plsc.load_gather(evb, [z16 + h])
                kb[i, sl] = (vb[i, sl] + rvb[i, sl]) * eh

    def zden(i, carry):
        denf[pl.ds(i * 16, 16)] = jnp.zeros((16,), jnp.float32)
        return carry
    lax.fori_loop(0, DEN_W // 16, zden, 0)

    fire_idx(0, 0)
    wait_idx(0, 0)
    fire_gath(0)
    fire_idx(1, 1)

    def step(t, carry):
        g0 = 2 * t

        @pl.when(g0 >= 1)
        def _():
            wait_store(g0 - 1, 1)
        wait_idx(g0 + 1, 1)
        fire_gath(1)
        wait_gath(0)
        compute(0)
        fire_store(g0, 0)
        fire_idx(g0 + 2, 0)
        wait_idx(g0 + 2, 0)
        wait_store(g0, 0)
        fire_gath(0)
        wait_gath(1)
        compute(1)
        fire_store(g0 + 1, 1)
        fire_idx(g0 + 3, 1)
        return carry

    lax.fori_loop(0, (NFULL - 1) // 2, step, 0)

    wait_store(NFULL - 2, 1)
    wait_gath(0)
    compute(0)
    fire_store(NFULL - 1, 0)
    wait_idx(NFULL, 1)
    wait_store(NFULL - 1, 0)

    pltpu.sync_copy(denf, denp_h.at[wid])


def _sc_edge(srcp, dstp, ikp, iqp, ivp, irp, kn, rtek, qr, vr, rtev):
    mesh = plsc.VectorSubcoreMesh(core_axis_name="c", subcore_axis_name="s")
    ib = [pltpu.VMEM((CH,), jnp.int32) for _ in range(12)]
    db = [pltpu.VMEM((CH, HID), jnp.float32) for _ in range(10)]
    f = functools.partial(
        pl.kernel,
        out_type=(jax.ShapeDtypeStruct((N_EDGES, HID), jnp.float32),
                  jax.ShapeDtypeStruct((NW, DEN_W), jnp.float32)),
        mesh=mesh,
        scratch_types=ib + db + [
            pltpu.VMEM((DEN_W,), jnp.float32),
            pltpu.VMEM((16,), jnp.float32),
            pltpu.VMEM((16,), jnp.float32),
            pltpu.SemaphoreType.DMA,
            pltpu.SemaphoreType.DMA,
            pltpu.SemaphoreType.DMA,
            pltpu.SemaphoreType.DMA,
            pltpu.SemaphoreType.DMA,
            pltpu.SemaphoreType.DMA,
        ],
        compiler_params=pltpu.CompilerParams(needs_layout_passes=False),
    )(_sc_edge_body)
    return f(srcp, dstp, ikp, iqp, ivp, irp, kn, rtek, qr, vr, rtev)


def _sc_scatter_body(m_h, dst_h, num_h, shared, tmp,
                     mab0, mab1, dstb0, dstb1, idxb0, idxb1, semL0, semL1):
    cid = lax.axis_index("c")
    sid = lax.axis_index("s")
    off = cid * HALF
    base0 = sid * EPS
    mab = (mab0, mab1)
    dstb = (dstb0, dstb1)
    idxb = (idxb0, idxb1)
    semL = (semL0, semL1)

    def fire_loads(c, p):
        @pl.when(c < NCHS)
        def _():
            pltpu.async_copy(dst_h.at[pl.ds(base0 + c * SCH, SCH)], dstb[p], semL[p])
            pltpu.async_copy(m_h.at[pl.ds(base0 + c * SCH, SCH)], mab[p], semL[p])

    def wait_loads(c, p):
        pltpu.make_async_copy(dst_h.at[pl.ds(base0 + c * SCH, SCH)], dstb[p], semL[p]).wait()
        pltpu.make_async_copy(m_h.at[pl.ds(base0 + c * SCH, SCH)], mab[p], semL[p]).wait()

    def do_scatter(p):
        for j in range(SCH // 16):
            sl = pl.ds(j * 16, 16)
            local = dstb[p][sl] - off
            ok = (local >= 0) & (local < HALF)
            idxb[p][sl] = jnp.where(ok, local, HALF)
        pltpu.sync_copy(mab[p], shared.at[idxb[p]], add=True)

    def zrow(i, carry):
        for j in range(HID // 16):
            tmp[i, pl.ds(j * 16, 16)] = jnp.zeros((16,), jnp.float32)
        return carry

    lax.fori_loop(0, ZROWS, zrow, 0)
    pltpu.sync_copy(tmp.at[pl.ds(0, ZROWS)], shared.at[pl.ds(sid * ZROWS, ZROWS)])
    plsc.subcore_barrier()

    fire_loads(0, 0)
    fire_loads(1, 1)

    def step(t, carry):
        c0 = 2 * t
        wait_loads(c0, 0)
        fire_loads(c0 + 2, 0)
        do_scatter(0)
        wait_loads(c0 + 1, 1)
        fire_loads(c0 + 3, 1)
        do_scatter(1)
        return carry

    lax.fori_loop(0, NCHS // 2, step, 0)
    plsc.subcore_barrier()
    pltpu.sync_copy(shared.at[pl.ds(sid * DROWS, DROWS)], tmp.at[pl.ds(0, DROWS)])
    pltpu.sync_copy(tmp.at[pl.ds(0, DROWS)], num_h.at[pl.ds(off + sid * DROWS, DROWS)])


def _sc_scatter(m, dst):
    mesh = plsc.VectorSubcoreMesh(core_axis_name="c", subcore_axis_name="s")
    f = functools.partial(
        pl.kernel,
        out_type=jax.ShapeDtypeStruct((NPAD, HID), jnp.float32),
        mesh=mesh,
        scratch_types=[
            pltpu.VMEM_SHARED((ACC_ROWS, HID), jnp.float32),
            pltpu.VMEM((ZROWS, HID), jnp.float32),
            pltpu.VMEM((SCH, HID), jnp.float32),
            pltpu.VMEM((SCH, HID), jnp.float32),
            pltpu.VMEM((SCH,), jnp.int32),
            pltpu.VMEM((SCH,), jnp.int32),
            pltpu.VMEM((SCH,), jnp.int32),
            pltpu.VMEM((SCH,), jnp.int32),
            pltpu.SemaphoreType.DMA,
            pltpu.SemaphoreType.DMA,
        ],
        compiler_params=pltpu.CompilerParams(needs_layout_passes=False),
    )(_sc_scatter_body)
    return f(m, dst)


def kernel(node_feature, node_type, edge_time, edge_index, edge_type,
           adapt_W, adapt_b, Wk, bk, Wq, bq, Wv, bv, Wa, ba,
           rel_pri, rel_att, rel_msg, skip, rte_W, rte_b, cls_W, cls_b):
    nt = node_type.astype(jnp.int32)
    src = edge_index[0].astype(jnp.int32)
    dst = edge_index[1].astype(jnp.int32)
    et = edge_type.astype(jnp.int32)
    tm = edge_time.astype(jnp.int32)

    stype = nt[src]
    ik = stype * 128 + tm
    iq = dst * N_REL + et
    iv = src * N_REL + et
    ir = ik * N_REL + et

    def padE(a):
        return jnp.pad(a, (0, EPAD - N_EDGES))

    srcp, dstp, ikp, iqp, ivp, irp = (padE(a) for a in (src, dst, ik, iq, iv, ir))

    oh = (nt[:, None] == jnp.arange(N_TYPES, dtype=jnp.int32)[None, :]).astype(jnp.float32)
    oh_p = jnp.pad(oh, ((0, NPAD - N_NODES), (0, 0)))
    nf_p = jnp.pad(node_feature, ((0, NPAD - N_NODES), (0, 0)))

    pe = _pe_table()
    sig = jax.nn.sigmoid(skip)

    x = _adapt(nf_p, oh_p, adapt_W, adapt_b)

    for l in range(N_LAYERS):
        scale = jnp.repeat(rel_pri[l], D_K, axis=-1) / math.sqrt(D_K)
        Aatt = _blockdiag(jnp.swapaxes(rel_att[l], -1, -2)) * scale[:, None, :]
        Amsg = _blockdiag(rel_msg[l])

        rtek, rtev = _tables(pe, rte_W[l], rte_b[l][None, :], Wk[l], Wv[l], Amsg)
        kn, qr, vr = _nodepre(x, oh_p, Wk[l], bk[l], Wq[l], bq[l], Wv[l], bv[l], Aatt, Amsg)

        m, denp = _sc_edge(srcp, dstp, ikp, iqp, ivp, irp, kn, rtek, qr, vr, rtev)
        num = _sc_scatter(m, dst)
        x = _update(num, denp.reshape(NW, NPAD, N_HEADS), x, oh_p,
                    Wa[l], ba[l], sig[l][:, None])

    out = _cls(x, cls_W, cls_b[None, :])
    return out[:N_NODES]


# speedup vs baseline: 1.4789x; 1.4789x over previous
import functools
import math

import numpy as np

import jax
import jax.numpy as jnp
from jax import lax
from jax.experimental import pallas as pl
from jax.experimental.pallas import tpu as pltpu
from jax.experimental.pallas import tpu_sc as plsc

N_NODES = 10000
N_EDGES = 320000
IN_DIM = 128
HID = 128
N_TYPES = 4
N_REL = 8
N_HEADS = 8
D_K = 16
N_LAYERS = 2
N_OUT = 16

NPAD = 10240
NB = 256
EPAD = N_EDGES + 128

NW = 32
EPW = N_EDGES // NW
CH = 16
NFULL = 625
DEN_W = NPAD * N_HEADS

HALF = NPAD // 2
ACC_ROWS = HALF + 128
ZROWS = ACC_ROWS // 16
DROWS = HALF // 16
SCH = 80
EPS = N_EDGES // 16
NCHS = EPS // SCH

def _pe_table():
    t = np.arange(128, dtype=np.float64)[:, None]
    div = np.exp(np.arange(0, HID, 2, dtype=np.float64) * (-(math.log(10000.0) / HID)))
    ang = t * div[None, :]
    pe = np.stack([np.sin(ang), np.cos(ang)], axis=-1).reshape(128, HID)
    return jnp.asarray(pe / math.sqrt(HID), jnp.float32)


def _blockdiag(a):
    out = jnp.zeros(a.shape[:-3] + (HID, HID), jnp.float32)
    for h in range(N_HEADS):
        out = out.at[..., h * D_K:(h + 1) * D_K, h * D_K:(h + 1) * D_K].set(a[..., h, :, :])
    return out


def _adapt_body(nf, oh, W, b, out):
    x = nf[...]
    ohv = oh[...]
    acc = jnp.zeros((NB, HID), jnp.float32)
    for t in range(N_TYPES):
        y = jnp.tanh(jnp.dot(x, W[t], preferred_element_type=jnp.float32)
                     + b[t][None, :])
        acc = acc + ohv[:, t][:, None] * y
    out[...] = acc


def _adapt(nf_p, oh_p, adapt_W, adapt_b):
    return pl.pallas_call(
        _adapt_body,
        grid=(NPAD // NB,),
        in_specs=[
            pl.BlockSpec((NB, IN_DIM), lambda i: (i, 0)),
            pl.BlockSpec((NB, N_TYPES), lambda i: (i, 0)),
            pl.BlockSpec((N_TYPES, IN_DIM, HID), lambda i: (0, 0, 0)),
            pl.BlockSpec((N_TYPES, HID), lambda i: (0, 0)),
        ],
        out_specs=pl.BlockSpec((NB, HID), lambda i: (i, 0)),
        out_shape=jax.ShapeDtypeStruct((NPAD, HID), jnp.float32),
    )(nf_p, oh_p, adapt_W, adapt_b)


def _tables_body(pe, rw, rb, Wk, Wv, Amsg, rk_out, rv_out):
    rte = jnp.dot(pe[...], rw[...], preferred_element_type=jnp.float32) + rb[0][None, :]
    for t in range(N_TYPES):
        rk_out[t] = jnp.dot(rte, Wk[t], preferred_element_type=jnp.float32)
        rv = jnp.dot(rte, Wv[t], preferred_element_type=jnp.float32)
        for r in range(N_REL):
            rv_out[t, :, r, :] = jnp.dot(rv, Amsg[r], preferred_element_type=jnp.float32)


def _tables(pe, rw, rb2, Wk_l, Wv_l, Amsg):
    rk, rv = pl.pallas_call(
        _tables_body,
        grid=(1,),
        in_specs=[
            pl.BlockSpec((128, HID), lambda i: (0, 0)),
            pl.BlockSpec((HID, HID), lambda i: (0, 0)),
            pl.BlockSpec((1, HID), lambda i: (0, 0)),
            pl.BlockSpec((N_TYPES, HID, HID), lambda i: (0, 0, 0)),
            pl.BlockSpec((N_TYPES, HID, HID), lambda i: (0, 0, 0)),
            pl.BlockSpec((N_REL, HID, HID), lambda i: (0, 0, 0)),
        ],
        out_specs=[
            pl.BlockSpec((N_TYPES, 128, HID), lambda i: (0, 0, 0)),
            pl.BlockSpec((N_TYPES, 128, N_REL, HID), lambda i: (0, 0, 0, 0)),
        ],
        out_shape=[
            jax.ShapeDtypeStruct((N_TYPES, 128, HID), jnp.float32),
            jax.ShapeDtypeStruct((N_TYPES, 128, N_REL, HID), jnp.float32),
        ],
    )(pe, rw, rb2, Wk_l, Wv_l, Amsg)
    return rk.reshape(N_TYPES * 128, HID), rv.reshape(N_TYPES * 128 * N_REL, HID)


def _nodepre_body(x, oh, Wk, bk, Wq, bq, Wv, bv, Aatt, Amsg, kn_out, qr_out, vr_out):
    xv = x[...]
    ohv = oh[...]

    def tlin(W, b):
        acc = jnp.zeros((NB, HID), jnp.float32)
        for t in range(N_TYPES):
            acc = acc + ohv[:, t][:, None] * (
                jnp.dot(xv, W[t], preferred_element_type=jnp.float32)
                + b[t][None, :])
        return acc

    K = tlin(Wk, bk)
    Q = tlin(Wq, bq)
    V = tlin(Wv, bv)
    kn_out[...] = K
    for r in range(N_REL):
        qr_out[:, r, :] = jnp.dot(Q, Aatt[r], preferred_element_type=jnp.float32)
        vr_out[:, r, :] = jnp.dot(V, Amsg[r], preferred_element_type=jnp.float32)


def _nodepre(x, oh_p, Wk_l, bk_l, Wq_l, bq_l, Wv_l, bv_l, Aatt, Amsg):
    w3 = pl.BlockSpec((N_TYPES, HID, HID), lambda i: (0, 0, 0))
    b2 = pl.BlockSpec((N_TYPES, HID), lambda i: (0, 0))
    r3 = pl.BlockSpec((N_REL, HID, HID), lambda i: (0, 0, 0))
    kn, qr, vr = pl.pallas_call(
        _nodepre_body,
        grid=(NPAD // NB,),
        in_specs=[
            pl.BlockSpec((NB, HID), lambda i: (i, 0)),
            pl.BlockSpec((NB, N_TYPES), lambda i: (i, 0)),
            w3, b2, w3, b2, w3, b2, r3, r3,
        ],
        out_specs=[
            pl.BlockSpec((NB, HID), lambda i: (i, 0)),
            pl.BlockSpec((NB, N_REL, HID), lambda i: (i, 0, 0)),
            pl.BlockSpec((NB, N_REL, HID), lambda i: (i, 0, 0)),
        ],
        out_shape=[
            jax.ShapeDtypeStruct((NPAD, HID), jnp.float32),
            jax.ShapeDtypeStruct((NPAD, N_REL, HID), jnp.float32),
            jax.ShapeDtypeStruct((NPAD, N_REL, HID), jnp.float32),
        ],
    )(x, oh_p, Wk_l, bk_l, Wq_l, bq_l, Wv_l, bv_l, Aatt, Amsg)
    return kn, qr.reshape(NPAD * N_REL, HID), vr.reshape(NPAD * N_REL, HID)


def _update_body(num_r, denp, x, oh, Wa, ba, sig, out):
    den8 = jnp.sum(denp[...], axis=0)
    i1 = lax.broadcasted_iota(jnp.int32, (N_HEADS, HID), 0)
    j1 = lax.broadcasted_iota(jnp.int32, (N_HEADS, HID), 1)
    Sm = (i1 == (j1 // D_K)).astype(jnp.float32)
    denx = jnp.dot(den8, Sm, preferred_element_type=jnp.float32) + 1e-16
    agg = num_r[...] / denx
    g = jax.nn.gelu(agg)
    ohv = oh[...]
    trans = jnp.zeros((NB, HID), jnp.float32)
    for t in range(N_TYPES):
        trans = trans + ohv[:, t][:, None] * (
            jnp.dot(g, Wa[t], preferred_element_type=jnp.float32)
            + ba[t][None, :])
    alph = jnp.dot(ohv, sig[...], preferred_element_type=jnp.float32)
    out[...] = trans * alph + x[...] * (1.0 - alph)


def _update(num, denp, x, oh_p, Wa_l, ba_l, sig_col):
    return pl.pallas_call(
        _update_body,
        grid=(NPAD // NB,),
        in_specs=[
            pl.BlockSpec((NB, HID), lambda i: (i, 0)),
            pl.BlockSpec((NW, NB, N_HEADS), lambda i: (0, i, 0)),
            pl.BlockSpec((NB, HID), lambda i: (i, 0)),
            pl.BlockSpec((NB, N_TYPES), lambda i: (i, 0)),
            pl.BlockSpec((N_TYPES, HID, HID), lambda i: (0, 0, 0)),
            pl.BlockSpec((N_TYPES, HID), lambda i: (0, 0)),
            pl.BlockSpec((N_TYPES, 1), lambda i: (0, 0)),
        ],
        out_specs=pl.BlockSpec((NB, HID), lambda i: (i, 0)),
        out_shape=jax.ShapeDtypeStruct((NPAD, HID), jnp.float32),
    )(num, denp, x, oh_p, Wa_l, ba_l, sig_col)


def _cls_body(x, W, b, out):
    logits = jnp.dot(x[...], W[...], preferred_element_type=jnp.float32) + b[0][None, :]
    m = jnp.max(logits, axis=-1, keepdims=True)
    z = logits - m
    out[...] = z - jnp.log(jnp.sum(jnp.exp(z), axis=-1, keepdims=True))


def _cls(x, cls_W, cls_b2):
    return pl.pallas_call(
        _cls_body,
        grid=(NPAD // NB,),
        in_specs=[
            pl.BlockSpec((NB, HID), lambda i: (i, 0)),
            pl.BlockSpec((HID, N_OUT), lambda i: (0, 0)),
            pl.BlockSpec((1, N_OUT), lambda i: (0, 0)),
        ],
        out_specs=pl.BlockSpec((NB, N_OUT), lambda i: (i, 0)),
        out_shape=jax.ShapeDtypeStruct((NPAD, N_OUT), jnp.float32),
    )(x, cls_W, cls_b2)


def _sc_edge_body(src_h, dst_h, ik_h, iq_h, iv_h, ir_h,
                  kn_h, rtek_h, qr_h, vr_h, rtev_h,
                  m_h, denp_h,
                  srcb0, dstb0, ikb0, iqb0, ivb0, irb0,
                  srcb1, dstb1, ikb1, iqb1, ivb1, irb1,
                  kb0, rkb0, qb0, vb0, rvb0,
                  kb1, rkb1, qb1, vb1, rvb1,
                  denf,
                  semI0, semI1, semG0, semG1, semS0, semS1):
    cid = lax.axis_index("c")
    sid = lax.axis_index("s")
    wid = sid * 2 + cid
    base0 = wid * EPW
    lane = lax.broadcasted_iota(jnp.int32, (16,), 0)

    idxs = ((srcb0, dstb0, ikb0, iqb0, ivb0, irb0),
            (srcb1, dstb1, ikb1, iqb1, ivb1, irb1))
    data = ((kb0, rkb0, qb0, vb0, rvb0),
            (kb1, rkb1, qb1, vb1, rvb1))
    semI = (semI0, semI1)
    semG = (semG0, semG1)
    semS = (semS0, semS1)
    ih = (src_h, dst_h, ik_h, iq_h, iv_h, ir_h)
    th = (kn_h, rtek_h, qr_h, vr_h, rtev_h)

    def fire_idx(c, p):
        for a in range(6):
            pltpu.async_copy(ih[a].at[pl.ds(base0 + c * CH, CH)], idxs[p][a], semI[p])

    def wait_idx(c, p):
        for a in range(6):
            pltpu.make_async_copy(ih[a].at[pl.ds(base0 + c * CH, CH)],
                                  idxs[p][a], semI[p]).wait()

    def gidx(p):
        return (idxs[p][0], idxs[p][2], idxs[p][3], idxs[p][4], idxs[p][5])

    def fire_gath(p):
        g = gidx(p)
        for a in range(5):
            pltpu.async_copy(th[a].at[g[a]], data[p][a], semG[p])

    def wait_gath(p):
        g = gidx(p)
        for a in range(5):
            pltpu.make_async_copy(th[a].at[g[a]], data[p][a], semG[p]).wait()

    def fire_store(c, p):
        pltpu.async_copy(data[p][0], m_h.at[pl.ds(base0 + c * CH, CH)], semS[p])

    def wait_store(c, p):
        pltpu.make_async_copy(data[p][0], m_h.at[pl.ds(base0 + c * CH, CH)], semS[p]).wait()

    hmask = lane < N_HEADS

    def compute(p):
        kb, rkb, qb, vb, rvb = data[p]
        dvec = idxs[p][1][pl.ds(0, 16)]
        for i in range(CH):
            s = []
            for h in range(N_HEADS):
                sl = pl.ds(h * D_K, 16)
                kq = (kb[i, sl] + rkb[i, sl]) * qb[i, sl]
                s.append(jnp.sum(kq))
            attv = jnp.zeros((16,), jnp.float32)
            for h in range(N_HEADS):
                attv = jnp.where(lane == h, s[h], attv)
            ev = jnp.exp(jnp.minimum(attv, 80.0))
            plsc.addupdate_scatter(denf, [dvec[i] * N_HEADS + lane], ev, mask=hmask)
            for h in range(N_HEADS):
                sl = pl.ds(h * D_K, 16)
                kb[i, sl] = (vb[i, sl] + rvb[i, sl]) * ev[h]

    def zden(i, carry):
        denf[pl.ds(i * 16, 16)] = jnp.zeros((16,), jnp.float32)
        return carry
    lax.fori_loop(0, DEN_W // 16, zden, 0)

    fire_idx(0, 0)
    wait_idx(0, 0)
    fire_gath(0)
    fire_idx(1, 1)

    def step(t, carry):
        g0 = 2 * t

        @pl.when(g0 >= 1)
        def _():
            wait_store(g0 - 1, 1)
        wait_idx(g0 + 1, 1)
        fire_gath(1)
        wait_gath(0)
        compute(0)
        fire_store(g0, 0)
        fire_idx(g0 + 2, 0)
        wait_idx(g0 + 2, 0)
        wait_store(g0, 0)
        fire_gath(0)
        wait_gath(1)
        compute(1)
        fire_store(g0 + 1, 1)
        fire_idx(g0 + 3, 1)
        return carry

    lax.fori_loop(0, (NFULL - 1) // 2, step, 0)

    wait_store(NFULL - 2, 1)
    wait_gath(0)
    compute(0)
    fire_store(NFULL - 1, 0)
    wait_idx(NFULL, 1)
    wait_store(NFULL - 1, 0)

    pltpu.sync_copy(denf, denp_h.at[wid])


def _sc_edge(srcp, dstp, ikp, iqp, ivp, irp, kn, rtek, qr, vr, rtev):
    mesh = plsc.VectorSubcoreMesh(core_axis_name="c", subcore_axis_name="s")
    ib = [pltpu.VMEM((CH,), jnp.int32) for _ in range(12)]
    db = [pltpu.VMEM((CH, HID), jnp.float32) for _ in range(10)]
    f = functools.partial(
        pl.kernel,
        out_type=(jax.ShapeDtypeStruct((N_EDGES, HID), jnp.float32),
                  jax.ShapeDtypeStruct((NW, DEN_W), jnp.float32)),
        mesh=mesh,
        scratch_types=ib + db + [
            pltpu.VMEM((DEN_W,), jnp.float32),
            pltpu.SemaphoreType.DMA,
            pltpu.SemaphoreType.DMA,
            pltpu.SemaphoreType.DMA,
            pltpu.SemaphoreType.DMA,
            pltpu.SemaphoreType.DMA,
            pltpu.SemaphoreType.DMA,
        ],
        compiler_params=pltpu.CompilerParams(needs_layout_passes=False),
    )(_sc_edge_body)
    return f(srcp, dstp, ikp, iqp, ivp, irp, kn, rtek, qr, vr, rtev)


def _sc_scatter_body(m_h, dst_h, num_h, shared, tmp,
                     mab0, mab1, dstb0, dstb1, idxb0, idxb1, semL0, semL1):
    cid = lax.axis_index("c")
    sid = lax.axis_index("s")
    off = cid * HALF
    base0 = sid * EPS
    mab = (mab0, mab1)
    dstb = (dstb0, dstb1)
    idxb = (idxb0, idxb1)
    semL = (semL0, semL1)

    def fire_loads(c, p):
        @pl.when(c < NCHS)
        def _():
            pltpu.async_copy(dst_h.at[pl.ds(base0 + c * SCH, SCH)], dstb[p], semL[p])
            pltpu.async_copy(m_h.at[pl.ds(base0 + c * SCH, SCH)], mab[p], semL[p])

    def wait_loads(c, p):
        pltpu.make_async_copy(dst_h.at[pl.ds(base0 + c * SCH, SCH)], dstb[p], semL[p]).wait()
        pltpu.make_async_copy(m_h.at[pl.ds(base0 + c * SCH, SCH)], mab[p], semL[p]).wait()

    def do_scatter(p):
        for j in range(SCH // 16):
            sl = pl.ds(j * 16, 16)
            local = dstb[p][sl] - off
            ok = (local >= 0) & (local < HALF)
            idxb[p][sl] = jnp.where(ok, local, HALF)
        pltpu.sync_copy(mab[p], shared.at[idxb[p]], add=True)

    def zrow(i, carry):
        for j in range(HID // 16):
            tmp[i, pl.ds(j * 16, 16)] = jnp.zeros((16,), jnp.float32)
        return carry

    lax.fori_loop(0, ZROWS, zrow, 0)
    pltpu.sync_copy(tmp.at[pl.ds(0, ZROWS)], shared.at[pl.ds(sid * ZROWS, ZROWS)])
    plsc.subcore_barrier()

    fire_loads(0, 0)
    fire_loads(1, 1)

    def step(t, carry):
        c0 = 2 * t
        wait_loads(c0, 0)
        fire_loads(c0 + 2, 0)
        do_scatter(0)
        wait_loads(c0 + 1, 1)
        fire_loads(c0 + 3, 1)
        do_scatter(1)
        return carry

    lax.fori_loop(0, NCHS // 2, step, 0)
    plsc.subcore_barrier()
    pltpu.sync_copy(shared.at[pl.ds(sid * DROWS, DROWS)], tmp.at[pl.ds(0, DROWS)])
    pltpu.sync_copy(tmp.at[pl.ds(0, DROWS)], num_h.at[pl.ds(off + sid * DROWS, DROWS)])


def _sc_scatter(m, dst):
    mesh = plsc.VectorSubcoreMesh(core_axis_name="c", subcore_axis_name="s")
    f = functools.partial(
        pl.kernel,
        out_type=jax.ShapeDtypeStruct((NPAD, HID), jnp.float32),
        mesh=mesh,
        scratch_types=[
            pltpu.VMEM_SHARED((ACC_ROWS, HID), jnp.float32),
            pltpu.VMEM((ZROWS, HID), jnp.float32),
            pltpu.VMEM((SCH, HID), jnp.float32),
            pltpu.VMEM((SCH, HID), jnp.float32),
            pltpu.VMEM((SCH,), jnp.int32),
            pltpu.VMEM((SCH,), jnp.int32),
            pltpu.VMEM((SCH,), jnp.int32),
            pltpu.VMEM((SCH,), jnp.int32),
            pltpu.SemaphoreType.DMA,
            pltpu.SemaphoreType.DMA,
        ],
        compiler_params=pltpu.CompilerParams(needs_layout_passes=False),
    )(_sc_scatter_body)
    return f(m, dst)


def kernel(node_feature, node_type, edge_time, edge_index, edge_type,
           adapt_W, adapt_b, Wk, bk, Wq, bq, Wv, bv, Wa, ba,
           rel_pri, rel_att, rel_msg, skip, rte_W, rte_b, cls_W, cls_b):
    nt = node_type.astype(jnp.int32)
    src = edge_index[0].astype(jnp.int32)
    dst = edge_index[1].astype(jnp.int32)
    et = edge_type.astype(jnp.int32)
    tm = edge_time.astype(jnp.int32)

    stype = nt[src]
    ik = stype * 128 + tm
    iq = dst * N_REL + et
    iv = src * N_REL + et
    ir = ik * N_REL + et

    def padE(a):
        return jnp.pad(a, (0, EPAD - N_EDGES))

    srcp, dstp, ikp, iqp, ivp, irp = (padE(a) for a in (src, dst, ik, iq, iv, ir))

    oh = (nt[:, None] == jnp.arange(N_TYPES, dtype=jnp.int32)[None, :]).astype(jnp.float32)
    oh_p = jnp.pad(oh, ((0, NPAD - N_NODES), (0, 0)))
    nf_p = jnp.pad(node_feature, ((0, NPAD - N_NODES), (0, 0)))

    pe = _pe_table()
    sig = jax.nn.sigmoid(skip)

    x = _adapt(nf_p, oh_p, adapt_W, adapt_b)

    for l in range(N_LAYERS):
        scale = jnp.repeat(rel_pri[l], D_K, axis=-1) / math.sqrt(D_K)
        Aatt = _blockdiag(jnp.swapaxes(rel_att[l], -1, -2)) * scale[:, None, :]
        Amsg = _blockdiag(rel_msg[l])

        rtek, rtev = _tables(pe, rte_W[l], rte_b[l][None, :], Wk[l], Wv[l], Amsg)
        kn, qr, vr = _nodepre(x, oh_p, Wk[l], bk[l], Wq[l], bq[l], Wv[l], bv[l], Aatt, Amsg)

        m, denp = _sc_edge(srcp, dstp, ikp, iqp, ivp, irp, kn, rtek, qr, vr, rtev)
        num = _sc_scatter(m, dst)
        x = _update(num, denp.reshape(NW, NPAD, N_HEADS), x, oh_p,
                    Wa[l], ba[l], sig[l][:, None])

    out = _cls(x, cls_W, cls_b[None, :])
    return out[:N_NODES]


# speedup vs baseline: 1.4938x; 1.0100x over previous
import functools
import math

import numpy as np

import jax
import jax.numpy as jnp
from jax import lax
from jax.experimental import pallas as pl
from jax.experimental.pallas import tpu as pltpu
from jax.experimental.pallas import tpu_sc as plsc

N_NODES = 10000
N_EDGES = 320000
IN_DIM = 128
HID = 128
N_TYPES = 4
N_REL = 8
N_HEADS = 8
D_K = 16
N_LAYERS = 2
N_OUT = 16

NPAD = 10240
NB = 256
EPAD = N_EDGES + 128

NW = 32
EPW = N_EDGES // NW
CH = 16
NFULL = 625
DEN_W = NPAD * N_HEADS

HALF = NPAD // 2
ACC_ROWS = HALF + 128
ZROWS = ACC_ROWS // 16
DROWS = HALF // 16
SCH = 80
EPS = N_EDGES // 16
NCHS = EPS // SCH

def _pe_table():
    t = np.arange(128, dtype=np.float64)[:, None]
    div = np.exp(np.arange(0, HID, 2, dtype=np.float64) * (-(math.log(10000.0) / HID)))
    ang = t * div[None, :]
    pe = np.stack([np.sin(ang), np.cos(ang)], axis=-1).reshape(128, HID)
    return jnp.asarray(pe / math.sqrt(HID), jnp.float32)


def _blockdiag(a):
    out = jnp.zeros(a.shape[:-3] + (HID, HID), jnp.float32)
    for h in range(N_HEADS):
        out = out.at[..., h * D_K:(h + 1) * D_K, h * D_K:(h + 1) * D_K].set(a[..., h, :, :])
    return out


def _adapt_body(nf, oh, W, b, out):
    x = nf[...]
    ohv = oh[...]
    acc = jnp.zeros((NB, HID), jnp.float32)
    for t in range(N_TYPES):
        y = jnp.tanh(jnp.dot(x, W[t], preferred_element_type=jnp.float32)
                     + b[t][None, :])
        acc = acc + ohv[:, t][:, None] * y
    out[...] = acc


def _adapt(nf_p, oh_p, adapt_W, adapt_b):
    return pl.pallas_call(
        _adapt_body,
        grid=(NPAD // NB,),
        in_specs=[
            pl.BlockSpec((NB, IN_DIM), lambda i: (i, 0)),
            pl.BlockSpec((NB, N_TYPES), lambda i: (i, 0)),
            pl.BlockSpec((N_TYPES, IN_DIM, HID), lambda i: (0, 0, 0)),
            pl.BlockSpec((N_TYPES, HID), lambda i: (0, 0)),
        ],
        out_specs=pl.BlockSpec((NB, HID), lambda i: (i, 0)),
        out_shape=jax.ShapeDtypeStruct((NPAD, HID), jnp.float32),
    )(nf_p, oh_p, adapt_W, adapt_b)


def _tables_body(pe, rw, rb, Wk, Wv, Amsg, rk_out, rv_out):
    rte = jnp.dot(pe[...], rw[...], preferred_element_type=jnp.float32) + rb[0][None, :]
    for t in range(N_TYPES):
        rk_out[t] = jnp.dot(rte, Wk[t], preferred_element_type=jnp.float32)
        rv = jnp.dot(rte, Wv[t], preferred_element_type=jnp.float32)
        for r in range(N_REL):
            rv_out[t, :, r, :] = jnp.dot(rv, Amsg[r], preferred_element_type=jnp.float32)


def _tables(pe, rw, rb2, Wk_l, Wv_l, Amsg):
    rk, rv = pl.pallas_call(
        _tables_body,
        grid=(1,),
        in_specs=[
            pl.BlockSpec((128, HID), lambda i: (0, 0)),
            pl.BlockSpec((HID, HID), lambda i: (0, 0)),
            pl.BlockSpec((1, HID), lambda i: (0, 0)),
            pl.BlockSpec((N_TYPES, HID, HID), lambda i: (0, 0, 0)),
            pl.BlockSpec((N_TYPES, HID, HID), lambda i: (0, 0, 0)),
            pl.BlockSpec((N_REL, HID, HID), lambda i: (0, 0, 0)),
        ],
        out_specs=[
            pl.BlockSpec((N_TYPES, 128, HID), lambda i: (0, 0, 0)),
            pl.BlockSpec((N_TYPES, 128, N_REL, HID), lambda i: (0, 0, 0, 0)),
        ],
        out_shape=[
            jax.ShapeDtypeStruct((N_TYPES, 128, HID), jnp.float32),
            jax.ShapeDtypeStruct((N_TYPES, 128, N_REL, HID), jnp.float32),
        ],
    )(pe, rw, rb2, Wk_l, Wv_l, Amsg)
    return rk.reshape(N_TYPES * 128, HID), rv.reshape(N_TYPES * 128 * N_REL, HID)


def _nodepre_body(x, oh, Wk, bk, Wq, bq, Wv, bv, Aatt, Amsg, kn_out, qr_out, vr_out):
    xv = x[...]
    ohv = oh[...]

    def tlin(W, b):
        acc = jnp.zeros((NB, HID), jnp.float32)
        for t in range(N_TYPES):
            acc = acc + ohv[:, t][:, None] * (
                jnp.dot(xv, W[t], preferred_element_type=jnp.float32)
                + b[t][None, :])
        return acc

    K = tlin(Wk, bk)
    Q = tlin(Wq, bq)
    V = tlin(Wv, bv)
    kn_out[...] = K
    for r in range(N_REL):
        qr_out[:, r, :] = jnp.dot(Q, Aatt[r], preferred_element_type=jnp.float32)
        vr_out[:, r, :] = jnp.dot(V, Amsg[r], preferred_element_type=jnp.float32)


def _nodepre(x, oh_p, Wk_l, bk_l, Wq_l, bq_l, Wv_l, bv_l, Aatt, Amsg):
    w3 = pl.BlockSpec((N_TYPES, HID, HID), lambda i: (0, 0, 0))
    b2 = pl.BlockSpec((N_TYPES, HID), lambda i: (0, 0))
    r3 = pl.BlockSpec((N_REL, HID, HID), lambda i: (0, 0, 0))
    kn, qr, vr = pl.pallas_call(
        _nodepre_body,
        grid=(NPAD // NB,),
        in_specs=[
            pl.BlockSpec((NB, HID), lambda i: (i, 0)),
            pl.BlockSpec((NB, N_TYPES), lambda i: (i, 0)),
            w3, b2, w3, b2, w3, b2, r3, r3,
        ],
        out_specs=[
            pl.BlockSpec((NB, HID), lambda i: (i, 0)),
            pl.BlockSpec((NB, N_REL, HID), lambda i: (i, 0, 0)),
            pl.BlockSpec((NB, N_REL, HID), lambda i: (i, 0, 0)),
        ],
        out_shape=[
            jax.ShapeDtypeStruct((NPAD, HID), jnp.float32),
            jax.ShapeDtypeStruct((NPAD, N_REL, HID), jnp.float32),
            jax.ShapeDtypeStruct((NPAD, N_REL, HID), jnp.float32),
        ],
    )(x, oh_p, Wk_l, bk_l, Wq_l, bq_l, Wv_l, bv_l, Aatt, Amsg)
    return kn, qr.reshape(NPAD * N_REL, HID), vr.reshape(NPAD * N_REL, HID)


def _update_body(num_r, denp, x, oh, Wa, ba, sig, out):
    den8 = jnp.sum(denp[...], axis=0)
    i1 = lax.broadcasted_iota(jnp.int32, (N_HEADS, HID), 0)
    j1 = lax.broadcasted_iota(jnp.int32, (N_HEADS, HID), 1)
    Sm = (i1 == (j1 // D_K)).astype(jnp.float32)
    denx = jnp.dot(den8, Sm, preferred_element_type=jnp.float32) + 1e-16
    agg = num_r[...] / denx
    g = jax.nn.gelu(agg)
    ohv = oh[...]
    trans = jnp.zeros((NB, HID), jnp.float32)
    for t in range(N_TYPES):
        trans = trans + ohv[:, t][:, None] * (
            jnp.dot(g, Wa[t], preferred_element_type=jnp.float32)
            + ba[t][None, :])
    alph = jnp.dot(ohv, sig[...], preferred_element_type=jnp.float32)
    out[...] = trans * alph + x[...] * (1.0 - alph)


def _update(num, denp, x, oh_p, Wa_l, ba_l, sig_col):
    return pl.pallas_call(
        _update_body,
        grid=(NPAD // NB,),
        in_specs=[
            pl.BlockSpec((NB, HID), lambda i: (i, 0)),
            pl.BlockSpec((NW, NB, N_HEADS), lambda i: (0, i, 0)),
            pl.BlockSpec((NB, HID), lambda i: (i, 0)),
            pl.BlockSpec((NB, N_TYPES), lambda i: (i, 0)),
            pl.BlockSpec((N_TYPES, HID, HID), lambda i: (0, 0, 0)),
            pl.BlockSpec((N_TYPES, HID), lambda i: (0, 0)),
            pl.BlockSpec((N_TYPES, 1), lambda i: (0, 0)),
        ],
        out_specs=pl.BlockSpec((NB, HID), lambda i: (i, 0)),
        out_shape=jax.ShapeDtypeStruct((NPAD, HID), jnp.float32),
    )(num, denp, x, oh_p, Wa_l, ba_l, sig_col)


def _cls_body(x, W, b, out):
    logits = jnp.dot(x[...], W[...], preferred_element_type=jnp.float32) + b[0][None, :]
    m = jnp.max(logits, axis=-1, keepdims=True)
    z = logits - m
    out[...] = z - jnp.log(jnp.sum(jnp.exp(z), axis=-1, keepdims=True))


def _cls(x, cls_W, cls_b2):
    return pl.pallas_call(
        _cls_body,
        grid=(NPAD // NB,),
        in_specs=[
            pl.BlockSpec((NB, HID), lambda i: (i, 0)),
            pl.BlockSpec((HID, N_OUT), lambda i: (0, 0)),
            pl.BlockSpec((1, N_OUT), lambda i: (0, 0)),
        ],
        out_specs=pl.BlockSpec((NB, N_OUT), lambda i: (i, 0)),
        out_shape=jax.ShapeDtypeStruct((NPAD, N_OUT), jnp.float32),
    )(x, cls_W, cls_b2)


def _tlin_block(xv, ohv, W, b):
    acc = jnp.zeros((NB, HID), jnp.float32)
    for t in range(N_TYPES):
        acc = acc + ohv[:, t][:, None] * (
            jnp.dot(xv, W[t], preferred_element_type=jnp.float32) + b[t][None, :])
    return acc


def _nodepre_block(xv, ohv, Wk, bk, Wq, bq, Wv, bv, Aatt, Amsg, kn_out, qr_out, vr_out):
    K = _tlin_block(xv, ohv, Wk, bk)
    Q = _tlin_block(xv, ohv, Wq, bq)
    V = _tlin_block(xv, ohv, Wv, bv)
    kn_out[...] = K
    for r in range(N_REL):
        qr_out[:, r, :] = jnp.dot(Q, Aatt[r], preferred_element_type=jnp.float32)
        vr_out[:, r, :] = jnp.dot(V, Amsg[r], preferred_element_type=jnp.float32)


def _update_block(num_r, denp, x, ohv, Wa, ba, sig):
    den8 = jnp.sum(denp[...], axis=0)
    i1 = lax.broadcasted_iota(jnp.int32, (N_HEADS, HID), 0)
    j1 = lax.broadcasted_iota(jnp.int32, (N_HEADS, HID), 1)
    Sm = (i1 == (j1 // D_K)).astype(jnp.float32)
    denx = jnp.dot(den8, Sm, preferred_element_type=jnp.float32) + 1e-16
    agg = num_r[...] / denx
    g = jax.nn.gelu(agg)
    trans = _tlin_block(g, ohv, Wa, ba)
    alph = jnp.dot(ohv, sig[...], preferred_element_type=jnp.float32)
    return trans * alph + x[...] * (1.0 - alph)


def _adapt_pre_body(nf, oh, W, b, Wk, bk, Wq, bq, Wv, bv, Aatt, Amsg,
                    x_out, kn_out, qr_out, vr_out):
    ohv = oh[...]
    nfv = nf[...]
    acc = jnp.zeros((NB, HID), jnp.float32)
    for t in range(N_TYPES):
        y = jnp.tanh(jnp.dot(nfv, W[t], preferred_element_type=jnp.float32) + b[t][None, :])
        acc = acc + ohv[:, t][:, None] * y
    x_out[...] = acc
    _nodepre_block(acc, ohv, Wk, bk, Wq, bq, Wv, bv, Aatt, Amsg, kn_out, qr_out, vr_out)


def _upd_pre_body(num, denp, x, oh, Wa, ba, sig, Wk, bk, Wq, bq, Wv, bv, Aatt, Amsg,
                  x_out, kn_out, qr_out, vr_out):
    ohv = oh[...]
    xn = _update_block(num, denp, x, ohv, Wa, ba, sig)
    x_out[...] = xn
    _nodepre_block(xn, ohv, Wk, bk, Wq, bq, Wv, bv, Aatt, Amsg, kn_out, qr_out, vr_out)


def _upd_cls_body(num, denp, x, oh, Wa, ba, sig, W, b, out):
    xn = _update_block(num, denp, x, oh[...], Wa, ba, sig)
    logits = jnp.dot(xn, W[...], preferred_element_type=jnp.float32) + b[0][None, :]
    mx = jnp.max(logits, axis=-1, keepdims=True)
    z = logits - mx
    out[...] = z - jnp.log(jnp.sum(jnp.exp(z), axis=-1, keepdims=True))


_B2H = pl.BlockSpec((NB, HID), lambda i: (i, 0))
_W3 = pl.BlockSpec((N_TYPES, HID, HID), lambda i: (0, 0, 0))
_B2 = pl.BlockSpec((N_TYPES, HID), lambda i: (0, 0))
_R3 = pl.BlockSpec((N_REL, HID, HID), lambda i: (0, 0, 0))
_PRE_OUT_SPECS = [
    pl.BlockSpec((NB, HID), lambda i: (i, 0)),
    pl.BlockSpec((NB, HID), lambda i: (i, 0)),
    pl.BlockSpec((NB, N_REL, HID), lambda i: (i, 0, 0)),
    pl.BlockSpec((NB, N_REL, HID), lambda i: (i, 0, 0)),
]
_PRE_OUT_SHAPE = [
    jax.ShapeDtypeStruct((NPAD, HID), jnp.float32),
    jax.ShapeDtypeStruct((NPAD, HID), jnp.float32),
    jax.ShapeDtypeStruct((NPAD, N_REL, HID), jnp.float32),
    jax.ShapeDtypeStruct((NPAD, N_REL, HID), jnp.float32),
]


def _adapt_pre(nf_p, oh_p, aW, ab, Wk_l, bk_l, Wq_l, bq_l, Wv_l, bv_l, Aatt, Amsg):
    x, kn, qr, vr = pl.pallas_call(
        _adapt_pre_body,
        grid=(NPAD // NB,),
        in_specs=[
            pl.BlockSpec((NB, IN_DIM), lambda i: (i, 0)),
            pl.BlockSpec((NB, N_TYPES), lambda i: (i, 0)),
            pl.BlockSpec((N_TYPES, IN_DIM, HID), lambda i: (0, 0, 0)),
            _B2, _W3, _B2, _W3, _B2, _W3, _B2, _R3, _R3,
        ],
        out_specs=_PRE_OUT_SPECS,
        out_shape=_PRE_OUT_SHAPE,
    )(nf_p, oh_p, aW, ab, Wk_l, bk_l, Wq_l, bq_l, Wv_l, bv_l, Aatt, Amsg)
    return x, kn, qr.reshape(NPAD * N_REL, HID), vr.reshape(NPAD * N_REL, HID)


def _upd_pre(num, denp, x, oh_p, Wa_l, ba_l, sig_col,
             Wk_l, bk_l, Wq_l, bq_l, Wv_l, bv_l, Aatt, Amsg):
    xn, kn, qr, vr = pl.pallas_call(
        _upd_pre_body,
        grid=(NPAD // NB,),
        in_specs=[
            _B2H,
            pl.BlockSpec((NW, NB, N_HEADS), lambda i: (0, i, 0)),
            _B2H,
            pl.BlockSpec((NB, N_TYPES), lambda i: (i, 0)),
            _W3, _B2,
            pl.BlockSpec((N_TYPES, 1), lambda i: (0, 0)),
            _W3, _B2, _W3, _B2, _W3, _B2, _R3, _R3,
        ],
        out_specs=_PRE_OUT_SPECS,
        out_shape=_PRE_OUT_SHAPE,
    )(num, denp, x, oh_p, Wa_l, ba_l, sig_col,
      Wk_l, bk_l, Wq_l, bq_l, Wv_l, bv_l, Aatt, Amsg)
    return xn, kn, qr.reshape(NPAD * N_REL, HID), vr.reshape(NPAD * N_REL, HID)


def _upd_cls(num, denp, x, oh_p, Wa_l, ba_l, sig_col, cls_W, cls_b2):
    return pl.pallas_call(
        _upd_cls_body,
        grid=(NPAD // NB,),
        in_specs=[
            _B2H,
            pl.BlockSpec((NW, NB, N_HEADS), lambda i: (0, i, 0)),
            _B2H,
            pl.BlockSpec((NB, N_TYPES), lambda i: (i, 0)),
            _W3, _B2,
            pl.BlockSpec((N_TYPES, 1), lambda i: (0, 0)),
            pl.BlockSpec((HID, N_OUT), lambda i: (0, 0)),
            pl.BlockSpec((1, N_OUT), lambda i: (0, 0)),
        ],
        out_specs=pl.BlockSpec((NB, N_OUT), lambda i: (i, 0)),
        out_shape=jax.ShapeDtypeStruct((NPAD, N_OUT), jnp.float32),
    )(num, denp, x, oh_p, Wa_l, ba_l, sig_col, cls_W, cls_b2)


def _sc_edge_body(src_h, dst_h, ik_h, iq_h, iv_h, ir_h,
                  kn_h, rtek_h, qr_h, vr_h, rtev_h,
                  m_h, denp_h,
                  srcb0, dstb0, ikb0, iqb0, ivb0, irb0,
                  srcb1, dstb1, ikb1, iqb1, ivb1, irb1,
                  kb0, rkb0, qb0, vb0, rvb0,
                  kb1, rkb1, qb1, vb1, rvb1,
                  denf,
                  semI0, semI1, semG0, semG1, semS0, semS1):
    cid = lax.axis_index("c")
    sid = lax.axis_index("s")
    wid = sid * 2 + cid
    base0 = wid * EPW
    lane = lax.broadcasted_iota(jnp.int32, (16,), 0)

    idxs = ((srcb0, dstb0, ikb0, iqb0, ivb0, irb0),
            (srcb1, dstb1, ikb1, iqb1, ivb1, irb1))
    data = ((kb0, rkb0, qb0, vb0, rvb0),
            (kb1, rkb1, qb1, vb1, rvb1))
    semI = (semI0, semI1)
    semG = (semG0, semG1)
    semS = (semS0, semS1)
    ih = (src_h, dst_h, ik_h, iq_h, iv_h, ir_h)
    th = (kn_h, rtek_h, qr_h, vr_h, rtev_h)

    def fire_idx(c, p):
        for a in range(6):
            pltpu.async_copy(ih[a].at[pl.ds(base0 + c * CH, CH)], idxs[p][a], semI[p])

    def wait_idx(c, p):
        for a in range(6):
            pltpu.make_async_copy(ih[a].at[pl.ds(base0 + c * CH, CH)],
                                  idxs[p][a], semI[p]).wait()

    def gidx(p):
        return (idxs[p][0], idxs[p][2], idxs[p][3], idxs[p][4], idxs[p][5])

    def fire_gath(p):
        g = gidx(p)
        for a in range(5):
            pltpu.async_copy(th[a].at[g[a]], data[p][a], semG[p])

    def wait_gath(p):
        g = gidx(p)
        for a in range(5):
            pltpu.make_async_copy(th[a].at[g[a]], data[p][a], semG[p]).wait()

    def fire_store(c, p):
        pltpu.async_copy(data[p][0], m_h.at[pl.ds(base0 + c * CH, CH)], semS[p])

    def wait_store(c, p):
        pltpu.make_async_copy(data[p][0], m_h.at[pl.ds(base0 + c * CH, CH)], semS[p]).wait()

    hmask = lane < N_HEADS

    def compute(p):
        kb, rkb, qb, vb, rvb = data[p]
        dvec = idxs[p][1][pl.ds(0, 16)]
        for i in range(CH):
            s = []
            for h in range(N_HEADS):
                sl = pl.ds(h * D_K, 16)
                kq = (kb[i, sl] + rkb[i, sl]) * qb[i, sl]
                s.append(jnp.sum(kq))
            attv = jnp.zeros((16,), jnp.float32)
            for h in range(N_HEADS):
                attv = jnp.where(lane == h, s[h], attv)
            ev = jnp.exp(jnp.minimum(attv, 80.0))
            plsc.addupdate_scatter(denf, [dvec[i] * N_HEADS + lane], ev, mask=hmask)
            for h in range(N_HEADS):
                sl = pl.ds(h * D_K, 16)
                kb[i, sl] = (vb[i, sl] + rvb[i, sl]) * ev[h]

    def zden(i, carry):
        denf[pl.ds(i * 16, 16)] = jnp.zeros((16,), jnp.float32)
        return carry
    lax.fori_loop(0, DEN_W // 16, zden, 0)

    fire_idx(0, 0)
    wait_idx(0, 0)
    fire_gath(0)
    fire_idx(1, 1)

    def step(t, carry):
        g0 = 2 * t

        @pl.when(g0 >= 1)
        def _():
            wait_store(g0 - 1, 1)
        wait_idx(g0 + 1, 1)
        fire_gath(1)
        wait_gath(0)
        compute(0)
        fire_store(g0, 0)
        fire_idx(g0 + 2, 0)
        wait_idx(g0 + 2, 0)
        wait_store(g0, 0)
        fire_gath(0)
        wait_gath(1)
        compute(1)
        fire_store(g0 + 1, 1)
        fire_idx(g0 + 3, 1)
        return carry

    lax.fori_loop(0, (NFULL - 1) // 2, step, 0)

    wait_store(NFULL - 2, 1)
    wait_gath(0)
    compute(0)
    fire_store(NFULL - 1, 0)
    wait_idx(NFULL, 1)
    wait_store(NFULL - 1, 0)

    pltpu.sync_copy(denf, denp_h.at[wid])


def _sc_edge(srcp, dstp, ikp, iqp, ivp, irp, kn, rtek, qr, vr, rtev):
    mesh = plsc.VectorSubcoreMesh(core_axis_name="c", subcore_axis_name="s")
    ib = [pltpu.VMEM((CH,), jnp.int32) for _ in range(12)]
    db = [pltpu.VMEM((CH, HID), jnp.float32) for _ in range(10)]
    f = functools.partial(
        pl.kernel,
        out_type=(jax.ShapeDtypeStruct((N_EDGES, HID), jnp.float32),
                  jax.ShapeDtypeStruct((NW, DEN_W), jnp.float32)),
        mesh=mesh,
        scratch_types=ib + db + [
            pltpu.VMEM((DEN_W,), jnp.float32),
            pltpu.SemaphoreType.DMA,
            pltpu.SemaphoreType.DMA,
            pltpu.SemaphoreType.DMA,
            pltpu.SemaphoreType.DMA,
            pltpu.SemaphoreType.DMA,
            pltpu.SemaphoreType.DMA,
        ],
        compiler_params=pltpu.CompilerParams(needs_layout_passes=False),
    )(_sc_edge_body)
    return f(srcp, dstp, ikp, iqp, ivp, irp, kn, rtek, qr, vr, rtev)


def _sc_scatter_body(m_h, dst_h, num_h, shared, tmp,
                     mab0, mab1, dstb0, dstb1, idxb0, idxb1, semL0, semL1):
    cid = lax.axis_index("c")
    sid = lax.axis_index("s")
    off = cid * HALF
    base0 = sid * EPS
    mab = (mab0, mab1)
    dstb = (dstb0, dstb1)
    idxb = (idxb0, idxb1)
    semL = (semL0, semL1)

    def fire_loads(c, p):
        @pl.when(c < NCHS)
        def _():
            pltpu.async_copy(dst_h.at[pl.ds(base0 + c * SCH, SCH)], dstb[p], semL[p])
            pltpu.async_copy(m_h.at[pl.ds(base0 + c * SCH, SCH)], mab[p], semL[p])

    def wait_loads(c, p):
        pltpu.make_async_copy(dst_h.at[pl.ds(base0 + c * SCH, SCH)], dstb[p], semL[p]).wait()
        pltpu.make_async_copy(m_h.at[pl.ds(base0 + c * SCH, SCH)], mab[p], semL[p]).wait()

    def do_scatter(p):
        for j in range(SCH // 16):
            sl = pl.ds(j * 16, 16)
            local = dstb[p][sl] - off
            ok = (local >= 0) & (local < HALF)
            idxb[p][sl] = jnp.where(ok, local, HALF)
        pltpu.sync_copy(mab[p], shared.at[idxb[p]], add=True)

    def zrow(i, carry):
        for j in range(HID // 16):
            tmp[i, pl.ds(j * 16, 16)] = jnp.zeros((16,), jnp.float32)
        return carry

    lax.fori_loop(0, ZROWS, zrow, 0)
    pltpu.sync_copy(tmp.at[pl.ds(0, ZROWS)], shared.at[pl.ds(sid * ZROWS, ZROWS)])
    plsc.subcore_barrier()

    fire_loads(0, 0)
    fire_loads(1, 1)

    def step(t, carry):
        c0 = 2 * t
        wait_loads(c0, 0)
        fire_loads(c0 + 2, 0)
        do_scatter(0)
        wait_loads(c0 + 1, 1)
        fire_loads(c0 + 3, 1)
        do_scatter(1)
        return carry

    lax.fori_loop(0, NCHS // 2, step, 0)
    plsc.subcore_barrier()
    pltpu.sync_copy(shared.at[pl.ds(sid * DROWS, DROWS)], tmp.at[pl.ds(0, DROWS)])
    pltpu.sync_copy(tmp.at[pl.ds(0, DROWS)], num_h.at[pl.ds(off + sid * DROWS, DROWS)])


def _sc_scatter(m, dst):
    mesh = plsc.VectorSubcoreMesh(core_axis_name="c", subcore_axis_name="s")
    f = functools.partial(
        pl.kernel,
        out_type=jax.ShapeDtypeStruct((NPAD, HID), jnp.float32),
        mesh=mesh,
        scratch_types=[
            pltpu.VMEM_SHARED((ACC_ROWS, HID), jnp.float32),
            pltpu.VMEM((ZROWS, HID), jnp.float32),
            pltpu.VMEM((SCH, HID), jnp.float32),
            pltpu.VMEM((SCH, HID), jnp.float32),
            pltpu.VMEM((SCH,), jnp.int32),
            pltpu.VMEM((SCH,), jnp.int32),
            pltpu.VMEM((SCH,), jnp.int32),
            pltpu.VMEM((SCH,), jnp.int32),
            pltpu.SemaphoreType.DMA,
            pltpu.SemaphoreType.DMA,
        ],
        compiler_params=pltpu.CompilerParams(needs_layout_passes=False),
    )(_sc_scatter_body)
    return f(m, dst)


def kernel(node_feature, node_type, edge_time, edge_index, edge_type,
           adapt_W, adapt_b, Wk, bk, Wq, bq, Wv, bv, Wa, ba,
           rel_pri, rel_att, rel_msg, skip, rte_W, rte_b, cls_W, cls_b):
    nt = node_type.astype(jnp.int32)
    src = edge_index[0].astype(jnp.int32)
    dst = edge_index[1].astype(jnp.int32)
    et = edge_type.astype(jnp.int32)
    tm = edge_time.astype(jnp.int32)

    stype = nt[src]
    ik = stype * 128 + tm
    iq = dst * N_REL + et
    iv = src * N_REL + et
    ir = ik * N_REL + et

    def padE(a):
        return jnp.pad(a, (0, EPAD - N_EDGES))

    srcp, dstp, ikp, iqp, ivp, irp = (padE(a) for a in (src, dst, ik, iq, iv, ir))

    oh = (nt[:, None] == jnp.arange(N_TYPES, dtype=jnp.int32)[None, :]).astype(jnp.float32)
    oh_p = jnp.pad(oh, ((0, NPAD - N_NODES), (0, 0)))
    nf_p = jnp.pad(node_feature, ((0, NPAD - N_NODES), (0, 0)))

    pe = _pe_table()
    sig = jax.nn.sigmoid(skip)

    def layer_consts(l):
        scale = jnp.repeat(rel_pri[l], D_K, axis=-1) / math.sqrt(D_K)
        Aatt = _blockdiag(jnp.swapaxes(rel_att[l], -1, -2)) * scale[:, None, :]
        Amsg = _blockdiag(rel_msg[l])
        return Aatt, Amsg

    Aatt0, Amsg0 = layer_consts(0)
    Aatt1, Amsg1 = layer_consts(1)

    rtek0, rtev0 = _tables(pe, rte_W[0], rte_b[0][None, :], Wk[0], Wv[0], Amsg0)
    rtek1, rtev1 = _tables(pe, rte_W[1], rte_b[1][None, :], Wk[1], Wv[1], Amsg1)

    x0, kn0, qr0, vr0 = _adapt_pre(nf_p, oh_p, adapt_W, adapt_b,
                                   Wk[0], bk[0], Wq[0], bq[0], Wv[0], bv[0], Aatt0, Amsg0)
    m0, denp0 = _sc_edge(srcp, dstp, ikp, iqp, ivp, irp, kn0, rtek0, qr0, vr0, rtev0)
    num0 = _sc_scatter(m0, dst)
    x1, kn1, qr1, vr1 = _upd_pre(num0, denp0.reshape(NW, NPAD, N_HEADS), x0, oh_p,
                                 Wa[0], ba[0], sig[0][:, None],
                                 Wk[1], bk[1], Wq[1], bq[1], Wv[1], bv[1], Aatt1, Amsg1)
    m1, denp1 = _sc_edge(srcp, dstp, ikp, iqp, ivp, irp, kn1, rtek1, qr1, vr1, rtev1)
    num1 = _sc_scatter(m1, dst)
    out = _upd_cls(num1, denp1.reshape(NW, NPAD, N_HEADS), x1, oh_p,
                   Wa[1], ba[1], sig[1][:, None], cls_W, cls_b[None, :])
    return out[:N_NODES]
